# baseline (speedup 1.0000x reference)
import functools

import jax
import jax.numpy as jnp
from jax import lax
from jax.experimental import pallas as pl
from jax.experimental.pallas import tpu as pltpu
from jax.experimental.pallas import tpu_sc as plsc

_NUM_EMB = 100000
_EMB_DIM = 128
_WORDS = 16
_B = 4096 * 26
_LANES = 128
_ROWS = _B // _LANES
_NC, _NS = 2, 16
_NW = _NC * _NS
_RPW = _ROWS // _NW
_HALF = _RPW // 2
_SZW = 16

_SHIFTS = (4, 0, 12, 8, 20, 16, 28, 24)


def _sc_body(ids_hbm, tab_hbm, sz_hbm, out_hbm, idx_v, pk_v, sz_v, ob_v,
             sem_a, sem_b):
    wid = lax.axis_index("s") * _NC + lax.axis_index("c")
    pltpu.sync_copy(ids_hbm.at[wid], idx_v)

    def fire(lo, hi, sem):
        copies = []
        for j in range(lo, hi):
            copies.append(
                pltpu.async_copy(tab_hbm.at[idx_v.at[j]], pk_v.at[j], sem))
            copies.append(
                pltpu.async_copy(sz_hbm.at[idx_v.at[j]], sz_v.at[j], sem))
        return copies

    copies_a = fire(0, _HALF, sem_a)
    copies_b = fire(_HALF, _RPW, sem_b)

    lane = lax.iota(jnp.int32, 16)
    zero16 = jnp.zeros((16,), jnp.int32)
    one16 = jnp.ones((16,), jnp.int32)

    def chunk_body(c, carry):
        cc = jnp.full((16,), c, jnp.int32)

        def group_body(g, carry2):
            row = g * 16 + lane
            sv = plsc.load_gather(sz_v, [cc, row, zero16])
            zv = plsc.load_gather(sz_v, [cc, row, one16])
            ws = [
                plsc.load_gather(
                    pk_v, [cc, row, jnp.full((16,), m, jnp.int32)])
                for m in range(_WORDS)
            ]
            for m in range(_WORDS):
                w = ws[m]
                for e, sh in enumerate(_SHIFTS):
                    nib = (w >> sh) & 15
                    val = nib.astype(jnp.float32) * sv + zv
                    plsc.store_scatter(
                        ob_v, [row, jnp.full((16,), 8 * m + e, jnp.int32)],
                        val)
            return carry2

        lax.fori_loop(0, _LANES // 16, group_body, 0)
        pltpu.sync_copy(ob_v, out_hbm.at[wid, c])
        return carry

    for cpy in copies_a:
        cpy.wait()
    lax.fori_loop(0, _HALF, chunk_body, 0)
    for cpy in copies_b:
        cpy.wait()
    lax.fori_loop(_HALF, _RPW, chunk_body, 0)


@functools.lru_cache(maxsize=1)
def _make_sc_kernel():
    return pl.kernel(
        _sc_body,
        mesh=plsc.VectorSubcoreMesh(core_axis_name="c", subcore_axis_name="s"),
        out_type=jax.ShapeDtypeStruct((_NW, _RPW, _LANES, _EMB_DIM),
                                      jnp.float32),
        scratch_types=[
            pltpu.VMEM((_RPW, _LANES), jnp.int32),
            pltpu.VMEM((_RPW, _LANES, _WORDS), jnp.int32),
            pltpu.VMEM((_RPW, _LANES, _SZW), jnp.float32),
            pltpu.VMEM((_LANES, _EMB_DIM), jnp.float32),
            pltpu.SemaphoreType.DMA,
            pltpu.SemaphoreType.DMA,
        ],
        compiler_params=pltpu.CompilerParams(
            use_tc_tiling_on_sc=False, needs_layout_passes=False),
    )


def kernel(input_ids, weight_packed, scale, zero_point):
    bsz, seq = input_ids.shape
    ids_sm = input_ids.T.reshape(_NW, _RPW, _LANES)
    tab = lax.bitcast_convert_type(
        weight_packed.reshape(_NUM_EMB, _WORDS, 4), jnp.int32)
    zps = -(zero_point * scale)
    sz16 = jnp.concatenate(
        [scale, zps, jnp.zeros((_NUM_EMB, _SZW - 2), jnp.float32)], axis=1)
    out4 = _make_sc_kernel()(ids_sm, tab, sz16)
    return out4.reshape(seq, bsz, _EMB_DIM).transpose(1, 0, 2)
